# Initial kernel scaffold; baseline (speedup 1.0000x reference)
import jax
import jax.numpy as jnp
from jax.experimental import pallas as pl


def kernel(inputs, state, adj_mx, weight, biases):
    raise NotImplementedError("write your pallas kernel here")


# speedup vs baseline: 2.8283x; 2.8283x over previous
import functools

import jax
import jax.numpy as jnp
from jax.experimental import pallas as pl

_N = 1024
_F_IN = 64
_F_HID = 64
_F_OUT = 64
_BATCH = 32
_C = 4


def _gconv_body(xin_ref, st_ref, adj_ref, wa_ref, wb_ref, b_ref, out_ref):
    adj = adj_ref[...]
    wa = wa_ref[...]
    wb = wb_ref[...]
    xr = xin_ref[...].reshape(_C * _N, _F_IN)
    sr = st_ref[...].reshape(_C * _N, _F_HID)
    p = jnp.dot(xr, wa, preferred_element_type=jnp.float32)
    p = p + jnp.dot(sr, wb, preferred_element_type=jnp.float32)
    z2 = jnp.concatenate(
        [p[c * _N:(c + 1) * _N, 2 * _F_OUT:3 * _F_OUT] for c in range(_C)], axis=1)
    z1 = jnp.concatenate(
        [p[c * _N:(c + 1) * _N, _F_OUT:2 * _F_OUT] for c in range(_C)], axis=1)
    t = jnp.dot(adj, z2, preferred_element_type=jnp.float32)
    v = jnp.dot(adj, z1 + 2.0 * t, preferred_element_type=jnp.float32)
    bias = b_ref[...]
    for c in range(_C):
        out_ref[c] = (p[c * _N:(c + 1) * _N, 0:_F_OUT]
                      + v[:, c * _F_OUT:(c + 1) * _F_OUT] + bias)


@functools.partial(jax.jit, static_argnames=())
def kernel(inputs, state, adj_mx, weight, biases):
    batch = inputs.shape[0]
    xin = inputs.reshape(batch, _N, _F_IN)
    st = state.reshape(batch, _N, _F_HID)
    w0 = weight[0::3]
    w1 = weight[1::3]
    w2 = weight[2::3]
    wcat = jnp.concatenate([w0 - w2, w1, w2], axis=1)
    wa = wcat[:_F_IN]
    wb = wcat[_F_IN:]
    bias = biases.reshape(1, _F_OUT)

    out = pl.pallas_call(
        _gconv_body,
        grid=(batch // _C,),
        in_specs=[
            pl.BlockSpec((_C, _N, _F_IN), lambda i: (i, 0, 0)),
            pl.BlockSpec((_C, _N, _F_HID), lambda i: (i, 0, 0)),
            pl.BlockSpec((_N, _N), lambda i: (0, 0)),
            pl.BlockSpec((_F_IN, 3 * _F_OUT), lambda i: (0, 0)),
            pl.BlockSpec((_F_HID, 3 * _F_OUT), lambda i: (0, 0)),
            pl.BlockSpec((1, _F_OUT), lambda i: (0, 0)),
        ],
        out_specs=pl.BlockSpec((_C, _N, _F_OUT), lambda i: (i, 0, 0)),
        out_shape=jax.ShapeDtypeStruct((batch, _N, _F_OUT), jnp.float32),
    )(xin, st, adj_mx, wa, wb, bias)
    return out.reshape(batch, _N * _F_OUT)


# speedup vs baseline: 2.9433x; 1.0407x over previous
import functools

import jax
import jax.numpy as jnp
from jax.experimental import pallas as pl

_N = 1024
_F_IN = 64
_F_HID = 64
_F_OUT = 64
_BATCH = 32
_C = 8


def _gconv_body(xin_ref, st_ref, adj_ref, wa_ref, wb_ref, b_ref, out_ref):
    adj = adj_ref[...]
    wa = wa_ref[...]
    wb = wb_ref[...]
    xr = xin_ref[...].reshape(_C * _N, _F_IN)
    sr = st_ref[...].reshape(_C * _N, _F_HID)
    p = jnp.dot(xr, wa, preferred_element_type=jnp.float32)
    p = p + jnp.dot(sr, wb, preferred_element_type=jnp.float32)
    z2 = jnp.concatenate(
        [p[c * _N:(c + 1) * _N, 2 * _F_OUT:3 * _F_OUT] for c in range(_C)], axis=1)
    z1 = jnp.concatenate(
        [p[c * _N:(c + 1) * _N, _F_OUT:2 * _F_OUT] for c in range(_C)], axis=1)
    t = jnp.dot(adj, z2, preferred_element_type=jnp.float32)
    v = jnp.dot(adj, z1 + 2.0 * t, preferred_element_type=jnp.float32)
    bias = b_ref[...]
    for c in range(_C):
        out_ref[c] = (p[c * _N:(c + 1) * _N, 0:_F_OUT]
                      + v[:, c * _F_OUT:(c + 1) * _F_OUT] + bias)


@functools.partial(jax.jit, static_argnames=())
def kernel(inputs, state, adj_mx, weight, biases):
    batch = inputs.shape[0]
    xin = inputs.reshape(batch, _N, _F_IN)
    st = state.reshape(batch, _N, _F_HID)
    w0 = weight[0::3]
    w1 = weight[1::3]
    w2 = weight[2::3]
    wcat = jnp.concatenate([w0 - w2, w1, w2], axis=1)
    wa = wcat[:_F_IN]
    wb = wcat[_F_IN:]
    bias = biases.reshape(1, _F_OUT)

    out = pl.pallas_call(
        _gconv_body,
        grid=(batch // _C,),
        in_specs=[
            pl.BlockSpec((_C, _N, _F_IN), lambda i: (i, 0, 0)),
            pl.BlockSpec((_C, _N, _F_HID), lambda i: (i, 0, 0)),
            pl.BlockSpec((_N, _N), lambda i: (0, 0)),
            pl.BlockSpec((_F_IN, 3 * _F_OUT), lambda i: (0, 0)),
            pl.BlockSpec((_F_HID, 3 * _F_OUT), lambda i: (0, 0)),
            pl.BlockSpec((1, _F_OUT), lambda i: (0, 0)),
        ],
        out_specs=pl.BlockSpec((_C, _N, _F_OUT), lambda i: (i, 0, 0)),
        out_shape=jax.ShapeDtypeStruct((batch, _N, _F_OUT), jnp.float32),
    )(xin, st, adj_mx, wa, wb, bias)
    return out.reshape(batch, _N * _F_OUT)
